# baseline (speedup 1.0000x reference)
import functools

import jax
import jax.numpy as jnp
from jax import lax
from jax.experimental import pallas as pl
from jax.experimental.pallas import tpu as pltpu
from jax.experimental.pallas import tpu_sc as plsc

DIM = 64
NC = 2
NS = 16
NW = NC * NS
CW = 128
SUP = 32


def _sc_gather(x, table128):
    R, H = x.shape
    rows_per_w = R // NW
    n_sup = rows_per_w // SUP
    n_ch = SUP * H // CW
    assert n_sup % 2 == 0 and n_ch % 2 == 0 and (SUP * H) % CW == 0
    mesh = plsc.VectorSubcoreMesh(core_axis_name="c", subcore_axis_name="s")

    @functools.partial(
        pl.kernel,
        mesh=mesh,
        out_type=jax.ShapeDtypeStruct((R, H, DIM), jnp.float32),
        scratch_types=[
            pltpu.VMEM((2, SUP, H), jnp.int32),
            pltpu.VMEM((2, n_ch, CW), jnp.int32),
            pltpu.VMEM((2, CW, 128), jnp.float32),
            pltpu.VMEM((2, H, DIM), jnp.float32),
            pltpu.SemaphoreType.DMA((2,)),
            pltpu.SemaphoreType.DMA((2,)),
            pltpu.SemaphoreType.DMA((2,)),
        ],
    )
    def k(table_hbm, x_hbm, out_hbm, raw_v, flat_v, rows_v, out_v,
          sem_x, sem_g, sem_w):
        wid = lax.axis_index("s") * NC + lax.axis_index("c")
        xrow0 = wid * rows_per_w

        li = lax.iota(jnp.int32, 16)
        mlo = li < 8

        def start_x(s, rb):
            pltpu.async_copy(x_hbm.at[pl.ds(xrow0 + s * SUP, SUP)],
                             raw_v.at[rb], sem_x.at[rb])

        def wait_x(rb):
            pltpu.make_async_copy(x_hbm.at[pl.ds(0, SUP)],
                                  raw_v.at[rb], sem_x.at[rb]).wait()

        def compact_idx(rb, fb):
            for m in range(SUP * H // 16):
                q = 16 * m
                r, o = q // H, q % H
                cc, lane = q // CW, q % CW
                if o <= H - 16:
                    flat_v[fb, cc, pl.ds(lane, 16)] = \
                        raw_v[rb, r, pl.ds(o, 16)]
                else:
                    a = raw_v[rb, r, pl.ds(H - 16, 16)]
                    bv = raw_v[rb, r + 1, pl.ds(0, 16)]
                    hi = a.at[jnp.minimum(li + 8, 15)].get(
                        mode="promise_in_bounds")
                    lo = bv.at[jnp.maximum(li - 8, 0)].get(
                        mode="promise_in_bounds")
                    flat_v[fb, cc, pl.ds(lane, 16)] = jnp.where(mlo, hi, lo)

        def start_gather(fb, c, b):
            pltpu.async_copy(table_hbm.at[flat_v.at[fb, c]],
                             rows_v.at[b], sem_g.at[b])

        def wait_gather(b):
            pltpu.make_async_copy(table_hbm.at[pl.ds(0, CW)],
                                  rows_v.at[b], sem_g.at[b]).wait()

        def start_write(xr, z):
            pltpu.async_copy(out_v.at[z], out_hbm.at[xr], sem_w.at[z])

        def wait_write(z):
            pltpu.make_async_copy(out_hbm.at[0], out_v.at[z],
                                  sem_w.at[z]).wait()

        def copy_rows(b, z, src0, dst0, n8):
            def blk(i, cc):
                for kk in range(8):
                    for j in range(DIM // 16):
                        out_v[z, dst0 + 8 * i + kk, pl.ds(16 * j, 16)] = \
                            rows_v[b, src0 + 8 * i + kk, pl.ds(16 * j, 16)]
                return cc

            lax.fori_loop(0, n8, blk, 0)

        def chunk_tail(s, c, b):
            cg = s * n_ch + c
            p0 = cg * CW
            kk = p0 // H
            po = p0 - kk * H
            t = jnp.minimum(H - po, CW)
            comp = po + CW >= H
            cur = lax.rem(kk, 2)

            def body(z):
                copy_rows(b, z, 0, po, t >> 3)

                @pl.when(comp)
                def _():
                    start_write(xrow0 + kk, z)

                    @pl.when(kk >= 1)
                    def _():
                        wait_write(1 - z)

                    copy_rows(b, 1 - z, t, 0, (CW - t) >> 3)

            @pl.when(cur == 0)
            def _():
                body(0)

            @pl.when(cur == 1)
            def _():
                body(1)

        def sup_body(s, sb):
            @pl.when(s + 1 < n_sup)
            def _():
                wait_x(1 - sb)
                compact_idx(1 - sb, 1 - sb)

            @pl.when(s + 2 < n_sup)
            def _():
                start_x(s + 2, sb)

            def step(c, b):
                wait_gather(b)

                @pl.when(c + 1 < n_ch)
                def _():
                    start_gather(sb, c + 1, 1 - b)

                @pl.when((c + 1 >= n_ch) & (s + 1 < n_sup))
                def _():
                    start_gather(1 - sb, 0, 1 - b)

                chunk_tail(s, c, b)

            def chpair(p, cc):
                step(2 * p, 0)
                step(2 * p + 1, 1)
                return cc

            lax.fori_loop(0, n_ch // 2, chpair, 0)

        start_x(0, 0)
        wait_x(0)
        start_x(1, 1)
        compact_idx(0, 0)
        start_gather(0, 0, 0)

        def sup_pair(sp, cc):
            sup_body(2 * sp, 0)
            sup_body(2 * sp + 1, 1)
            return cc

        lax.fori_loop(0, n_sup // 2, sup_pair, 0)
        wait_write((rows_per_w - 1) % 2)

    return k(table128, x)


def kernel(x, embedding):
    table128 = jnp.pad(embedding, ((0, 0), (0, 128 - DIM)))
    return _sc_gather(x, table128)
